# baseline (speedup 1.0000x reference)
import functools

import jax
import jax.numpy as jnp
from jax import lax
from jax.experimental import pallas as pl
from jax.experimental.pallas import tpu as pltpu
from jax.experimental.pallas import tpu_sc as plsc

D_MODEL = 1024
SCALE = 32.0
CH = 8
LANES = 16
NBUF = 8
LOOK = 4
UNROLL = 4


def kernel(x, table, pos_enc):
    batch, seq = x.shape
    n_rows = batch * seq
    xf = x.astype(jnp.int32)

    info = plsc.get_sparse_core_info()
    nc, ns = info.num_cores, info.num_subcores
    nw = nc * ns
    seq_per_w = seq // nw
    sub_per_w = seq_per_w // CH
    n_chunks = batch * sub_per_w
    n_rounds = sub_per_w
    n_outer = n_rounds // 2

    mesh = plsc.VectorSubcoreMesh(core_axis_name="c", subcore_axis_name="s")

    @functools.partial(
        pl.kernel,
        mesh=mesh,
        out_type=jax.ShapeDtypeStruct((n_rows, D_MODEL), jnp.float32),
        scratch_types=[
            pltpu.VMEM((2, CH, D_MODEL), jnp.float32),
            pltpu.VMEM((NBUF, CH, D_MODEL), jnp.float32),
            pltpu.VMEM((batch, seq_per_w), jnp.int32),
        ]
        + [pltpu.SemaphoreType.DMA] * (2 * NBUF + 3),
    )
    def emb_kernel(x_hbm, tab_hbm, pos_hbm, out_hbm, pos_v, rows_v, idx_v,
                   *sems):
        gsem = sems[:NBUF]
        wsem = sems[NBUF:2 * NBUF]
        psem = sems[2 * NBUF:2 * NBUF + 2]
        isem = sems[2 * NBUF + 2]
        wid = lax.axis_index("c") * ns + lax.axis_index("s")
        s0 = wid * seq_per_w

        for b in range(batch):
            off = pl.multiple_of(s0, CH)
            pltpu.async_copy(x_hbm.at[b, pl.ds(off, seq_per_w)],
                             idx_v.at[b], isem)
        for b in range(batch):
            pltpu.make_async_copy(x_hbm.at[0, pl.ds(0, seq_per_w)],
                                  idx_v.at[b], isem).wait()

        def out_slice(ci):
            sub = lax.div(ci, batch)
            b = lax.rem(ci, batch)
            f = pl.multiple_of(b * seq + s0 + sub * CH, CH)
            return out_hbm.at[pl.ds(f, CH)]

        def idx_slice(ci):
            sub = lax.div(ci, batch)
            b = lax.rem(ci, batch)
            return idx_v.at[b, pl.ds(sub * CH, CH)]

        def start_pos(sub, pb):
            off = pl.multiple_of(s0 + sub * CH, CH)
            pltpu.async_copy(pos_hbm.at[pl.ds(off, CH)], pos_v.at[pb],
                             psem[pb])

        def start_gather(ci, bi):
            pltpu.async_copy(tab_hbm.at[idx_slice(ci)], rows_v.at[bi],
                             gsem[bi])

        def wait_gather(ci, bi):
            pltpu.make_async_copy(tab_hbm.at[idx_slice(ci)], rows_v.at[bi],
                                  gsem[bi]).wait()

        def wait_wb(ci, bi):
            pltpu.make_async_copy(rows_v.at[bi], out_slice(ci),
                                  wsem[bi]).wait()

        def wait_pos(pb):
            pltpu.make_async_copy(pos_hbm.at[pl.ds(0, CH)], pos_v.at[pb],
                                  psem[pb]).wait()

        def maybe_when(cond, fn):
            if isinstance(cond, bool):
                if cond:
                    fn()
            else:
                pl.when(cond)(fn)

        start_pos(0, 0)
        for ci in range(LOOK):
            start_gather(ci, ci)

        def round_body(outer, _):
            for h in range(2):
                r = outer * 2 + h
                ci0 = r * batch
                for k in range(batch):
                    ci = ci0 + k
                    bi = h * batch + k
                    nb = (bi + LOOK) % NBUF
                    gather_ok = True if h == 0 else (outer < n_outer - 1)
                    drain_ok = (outer > 0) if h == 0 else True

                    def drain(ci=ci, nb=nb):
                        wait_wb(ci - LOOK, nb)

                    def drain_and_gather(ci=ci, nb=nb, do_drain=drain_ok):
                        maybe_when(do_drain, lambda: wait_wb(ci - LOOK, nb))
                        start_gather(ci + LOOK, nb)

                    if isinstance(gather_ok, bool):
                        if gather_ok:
                            maybe_when(drain_ok, drain)
                            start_gather(ci + LOOK, nb)
                    else:
                        maybe_when(gather_ok, lambda ci=ci, nb=nb: (
                            wait_wb(ci - LOOK, nb),
                            start_gather(ci + LOOK, nb))[-1])
                    if k == 0:
                        wait_pos(h)
                        pos_ok = True if h == 0 else (outer < n_outer - 1)
                        maybe_when(pos_ok, lambda r=r, h=h:
                                   start_pos(r + 1, (h + 1) % 2))
                    wait_gather(ci, bi)

                    rv = rows_v.at[bi]
                    pv = pos_v.at[h]

                    def row_body(rr, _, rv=rv, pv=pv):
                        @plsc.parallel_loop(0, D_MODEL, step=LANES,
                                            unroll=UNROLL)
                        def _col(c):
                            sl = pl.ds(c, LANES)
                            rv[rr, sl] = rv[rr, sl] * SCALE + pv[rr, sl]

                        return 0

                    lax.fori_loop(0, CH, row_body, 0)
                    pltpu.async_copy(rv, out_slice(ci), wsem[bi])
            return 0

        lax.fori_loop(0, n_outer, round_body, 0)
        last = n_chunks - NBUF
        for j in range(NBUF):
            wait_wb(last + j, j)

    out = emb_kernel(xf, table, pos_enc)
    return out.reshape(batch, seq, D_MODEL)


# speedup vs baseline: 1.0082x; 1.0082x over previous
import functools

import jax
import jax.numpy as jnp
from jax import lax
from jax.experimental import pallas as pl
from jax.experimental.pallas import tpu as pltpu
from jax.experimental.pallas import tpu_sc as plsc

D_MODEL = 1024
SCALE = 32.0
CH = 8
LANES = 16
NGBUF = 8
NOBUF = 4
LOOK = 6
UNROLL = 4


def kernel(x, table, pos_enc):
    batch, seq = x.shape
    n_rows = batch * seq

    info = plsc.get_sparse_core_info()
    nc, ns = info.num_cores, info.num_subcores
    nw = nc * ns
    seq_per_w = seq // nw
    sub_per_w = seq_per_w // CH
    n_chunks = batch * sub_per_w
    n_rounds = sub_per_w
    n_outer = n_rounds // 2

    mesh = plsc.VectorSubcoreMesh(core_axis_name="c", subcore_axis_name="s")

    @functools.partial(
        pl.kernel,
        mesh=mesh,
        out_type=jax.ShapeDtypeStruct((n_rows, D_MODEL), jnp.float32),
        scratch_types=[
            pltpu.VMEM((2, CH, D_MODEL), jnp.float32),
            pltpu.VMEM((NGBUF, CH, D_MODEL), jnp.float32),
            pltpu.VMEM((NOBUF, CH, D_MODEL), jnp.float32),
            pltpu.VMEM((batch, seq_per_w), jnp.int32),
        ]
        + [pltpu.SemaphoreType.DMA] * (NGBUF + NOBUF + 3),
    )
    def emb_kernel(x_hbm, tab_hbm, pos_hbm, out_hbm, pos_v, rows_v, outs_v,
                   idx_v, *sems):
        gsem = sems[:NGBUF]
        wsem = sems[NGBUF:NGBUF + NOBUF]
        psem = sems[NGBUF + NOBUF:NGBUF + NOBUF + 2]
        isem = sems[NGBUF + NOBUF + 2]
        wid = lax.axis_index("c") * ns + lax.axis_index("s")
        s0 = wid * seq_per_w

        for b in range(batch):
            off = pl.multiple_of(s0, CH)
            pltpu.async_copy(x_hbm.at[b, pl.ds(off, seq_per_w)],
                             idx_v.at[b], isem)
        for b in range(batch):
            pltpu.make_async_copy(x_hbm.at[0, pl.ds(0, seq_per_w)],
                                  idx_v.at[b], isem).wait()

        def out_slice(ci):
            sub = lax.div(ci, batch)
            b = lax.rem(ci, batch)
            f = pl.multiple_of(b * seq + s0 + sub * CH, CH)
            return out_hbm.at[pl.ds(f, CH)]

        def idx_slice(ci):
            sub = lax.div(ci, batch)
            b = lax.rem(ci, batch)
            return idx_v.at[b, pl.ds(sub * CH, CH)]

        def start_pos(sub, pb):
            off = pl.multiple_of(s0 + sub * CH, CH)
            pltpu.async_copy(pos_hbm.at[pl.ds(off, CH)], pos_v.at[pb],
                             psem[pb])

        def start_gather(ci, bi):
            pltpu.async_copy(tab_hbm.at[idx_slice(ci)], rows_v.at[bi],
                             gsem[bi])

        def wait_gather(ci, bi):
            pltpu.make_async_copy(tab_hbm.at[idx_slice(ci)], rows_v.at[bi],
                                  gsem[bi]).wait()

        def wait_wb(ci, oi):
            pltpu.make_async_copy(outs_v.at[oi], out_slice(ci),
                                  wsem[oi]).wait()

        def wait_pos(pb):
            pltpu.make_async_copy(pos_hbm.at[pl.ds(0, CH)], pos_v.at[pb],
                                  psem[pb]).wait()

        def maybe_when(cond, fn):
            if isinstance(cond, bool):
                if cond:
                    fn()
            else:
                pl.when(cond)(fn)

        start_pos(0, 0)
        for ci in range(LOOK):
            start_gather(ci, ci)

        def round_body(outer, _):
            for h in range(2):
                r = outer * 2 + h
                ci0 = r * batch
                for k in range(batch):
                    ci = ci0 + k
                    bi = h * batch + k
                    oi = k
                    nb = (bi + LOOK) % NGBUF
                    if h == 0:
                        gather_ok = True if k < 2 else (outer < n_outer - 1)
                    else:
                        gather_ok = (outer < n_outer - 1)
                    maybe_when(gather_ok, lambda ci=ci, nb=nb:
                               start_gather(ci + LOOK, nb))
                    drain_ok = (outer > 0) if h == 0 else True
                    maybe_when(drain_ok, lambda ci=ci, oi=oi:
                               wait_wb(ci - NOBUF, oi))
                    if k == 0:
                        wait_pos(h)
                        pos_ok = True if h == 0 else (outer < n_outer - 1)
                        maybe_when(pos_ok, lambda r=r, h=h:
                                   start_pos(r + 1, (h + 1) % 2))
                    wait_gather(ci, bi)

                    rv = rows_v.at[bi]
                    ov = outs_v.at[oi]
                    pv = pos_v.at[h]

                    def row_body(rr, _, rv=rv, ov=ov, pv=pv):
                        @plsc.parallel_loop(0, D_MODEL, step=LANES,
                                            unroll=UNROLL)
                        def _col(c):
                            sl = pl.ds(c, LANES)
                            ov[rr, sl] = rv[rr, sl] * SCALE + pv[rr, sl]

                        return 0

                    lax.fori_loop(0, CH, row_body, 0)
                    pltpu.async_copy(ov, out_slice(ci), wsem[oi])
            return 0

        lax.fori_loop(0, n_outer, round_body, 0)
        last = n_chunks - NOBUF
        for j in range(NOBUF):
            wait_wb(last + j, j)

    out = emb_kernel(x.astype(jnp.int32), table, pos_enc)
    return out.reshape(batch, seq, D_MODEL)
